# Initial kernel scaffold; baseline (speedup 1.0000x reference)
import jax
import jax.numpy as jnp
from jax.experimental import pallas as pl


def kernel(x, W, b):
    raise NotImplementedError("write your pallas kernel here")


# speedup vs baseline: 1.1831x; 1.1831x over previous
import jax
import jax.numpy as jnp
from jax.experimental import pallas as pl

_K = 8


def _router_kernel(x_ref, w_ref, b_ref, idx_ref, val_ref):
    logits = jnp.dot(x_ref[...], w_ref[...], preferred_element_type=jnp.float32)
    logits = logits + b_ref[...]
    m = jnp.max(logits, axis=-1, keepdims=True)
    e = jnp.exp(logits - m)
    probs = e / jnp.sum(e, axis=-1, keepdims=True)
    num_e = probs.shape[-1]
    iota = jax.lax.broadcasted_iota(jnp.int32, probs.shape, 1)
    p = probs
    vals, idxs = [], []
    for _ in range(_K):
        mj = jnp.max(p, axis=-1, keepdims=True)
        ij = jnp.min(jnp.where(p == mj, iota, num_e), axis=-1, keepdims=True)
        vals.append(mj)
        idxs.append(ij)
        p = jnp.where(iota == ij, -1.0, p)
    idx_ref[...] = jnp.concatenate(idxs, axis=-1)
    val_ref[...] = jnp.concatenate(vals, axis=-1)


def kernel(x, W, b):
    B, S, D = x.shape
    E = W.shape[0]
    N = B * S
    xf = x.reshape(N, D)
    wt = W.T
    b2 = b.reshape(1, E)
    T = 1024
    idx, val = pl.pallas_call(
        _router_kernel,
        grid=(N // T,),
        in_specs=[
            pl.BlockSpec((T, D), lambda i: (i, 0)),
            pl.BlockSpec((D, E), lambda i: (0, 0)),
            pl.BlockSpec((1, E), lambda i: (0, 0)),
        ],
        out_specs=[
            pl.BlockSpec((T, _K), lambda i: (i, 0)),
            pl.BlockSpec((T, _K), lambda i: (i, 0)),
        ],
        out_shape=[
            jax.ShapeDtypeStruct((N, _K), jnp.int32),
            jax.ShapeDtypeStruct((N, _K), jnp.float32),
        ],
    )(xf, wt, b2)
    return idx.reshape(B, S, _K), val.reshape(B, S, _K)


# speedup vs baseline: 1.3869x; 1.1723x over previous
import jax
import jax.numpy as jnp
from jax.experimental import pallas as pl

_K = 8


def _router_kernel(x_ref, w_ref, b_ref, idx_ref, val_ref):
    logits = jnp.dot(x_ref[...], w_ref[...], preferred_element_type=jnp.float32)
    logits = logits + b_ref[...]
    m = jnp.max(logits, axis=-1, keepdims=True)
    e = jnp.exp(logits - m)
    s = jnp.sum(e, axis=-1, keepdims=True)
    num_e = e.shape[-1]
    iota = jax.lax.broadcasted_iota(jnp.int32, e.shape, 1)
    bits = jax.lax.bitcast_convert_type(e, jnp.int32)
    key = (bits & jnp.int32(~63)) | (jnp.int32(num_e - 1) - iota)
    vals, idxs = [], []
    for _ in range(_K):
        mk = jnp.max(key, axis=-1, keepdims=True)
        idxs.append(jnp.int32(num_e - 1) - (mk & jnp.int32(63)))
        vals.append(mk & jnp.int32(~63))
        key = jnp.where(key == mk, jnp.int32(0), key)
    idx_ref[...] = jnp.concatenate(idxs, axis=-1)
    topv = jax.lax.bitcast_convert_type(jnp.concatenate(vals, axis=-1), jnp.float32)
    val_ref[...] = topv / s


def kernel(x, W, b):
    B, S, D = x.shape
    E = W.shape[0]
    N = B * S
    xf = x.reshape(N, D)
    wt = W.T
    b2 = b.reshape(1, E)
    T = 1024
    idx, val = pl.pallas_call(
        _router_kernel,
        grid=(N // T,),
        in_specs=[
            pl.BlockSpec((T, D), lambda i: (i, 0)),
            pl.BlockSpec((D, E), lambda i: (0, 0)),
            pl.BlockSpec((1, E), lambda i: (0, 0)),
        ],
        out_specs=[
            pl.BlockSpec((T, _K), lambda i: (i, 0)),
            pl.BlockSpec((T, _K), lambda i: (i, 0)),
        ],
        out_shape=[
            jax.ShapeDtypeStruct((N, _K), jnp.int32),
            jax.ShapeDtypeStruct((N, _K), jnp.float32),
        ],
    )(xf, wt, b2)
    return idx.reshape(B, S, _K), val.reshape(B, S, _K)


# speedup vs baseline: 1.6174x; 1.1662x over previous
import jax
import jax.numpy as jnp
from jax.experimental import pallas as pl

_K = 8


def _router_kernel(x_ref, w_ref, b_ref, idx_ref, val_ref):
    logits = jnp.dot(x_ref[...], w_ref[...], preferred_element_type=jnp.float32)
    logits = logits + b_ref[...]
    m = jnp.max(logits, axis=-1, keepdims=True)
    e = jnp.exp(logits - m)
    s = jnp.sum(e, axis=-1, keepdims=True)
    num_e = e.shape[-1]
    iota = jax.lax.broadcasted_iota(jnp.int32, e.shape, 1)
    bits = jax.lax.bitcast_convert_type(e, jnp.int32)
    key = (bits & jnp.int32(~63)) | (jnp.int32(num_e - 1) - iota)
    keyf = jax.lax.bitcast_convert_type(key, jnp.float32)
    maxes = []
    for _ in range(_K):
        mkf = jnp.max(keyf, axis=-1, keepdims=True)
        maxes.append(mkf)
        keyf = jnp.where(keyf == mkf, jnp.float32(0.0), keyf)
    top_bits = jax.lax.bitcast_convert_type(
        jnp.concatenate(maxes, axis=-1), jnp.int32)
    idx_ref[...] = jnp.int32(num_e - 1) - (top_bits & jnp.int32(63))
    topv = jax.lax.bitcast_convert_type(top_bits & jnp.int32(~63), jnp.float32)
    val_ref[...] = topv / s


def kernel(x, W, b):
    B, S, D = x.shape
    E = W.shape[0]
    N = B * S
    xf = x.reshape(N, D)
    wt = W.T
    b2 = b.reshape(1, E)
    T = 1024
    idx, val = pl.pallas_call(
        _router_kernel,
        grid=(N // T,),
        in_specs=[
            pl.BlockSpec((T, D), lambda i: (i, 0)),
            pl.BlockSpec((D, E), lambda i: (0, 0)),
            pl.BlockSpec((1, E), lambda i: (0, 0)),
        ],
        out_specs=[
            pl.BlockSpec((T, _K), lambda i: (i, 0)),
            pl.BlockSpec((T, _K), lambda i: (i, 0)),
        ],
        out_shape=[
            jax.ShapeDtypeStruct((N, _K), jnp.int32),
            jax.ShapeDtypeStruct((N, _K), jnp.float32),
        ],
    )(xf, wt, b2)
    return idx.reshape(B, S, _K), val.reshape(B, S, _K)
